# Initial kernel scaffold; baseline (speedup 1.0000x reference)
import jax
import jax.numpy as jnp
from jax.experimental import pallas as pl


def kernel(x, pe_table):
    raise NotImplementedError("write your pallas kernel here")


# speedup vs baseline: 3.0186x; 3.0186x over previous
import jax
import jax.numpy as jnp
from jax.experimental import pallas as pl


def _copy_body(pe_ref, o_ref):
    o_ref[...] = pe_ref[...]


def kernel(x, pe_table):
    del x
    max_pos, emb_dim = pe_table.shape
    blk = 1024
    out = pl.pallas_call(
        _copy_body,
        grid=(max_pos // blk,),
        in_specs=[pl.BlockSpec((blk, emb_dim), lambda i: (i, 0))],
        out_specs=pl.BlockSpec((blk, emb_dim), lambda i: (i, 0)),
        out_shape=jax.ShapeDtypeStruct((max_pos, emb_dim), pe_table.dtype),
    )(pe_table)
    return out[None]
